# baseline (speedup 1.0000x reference)
import functools

import jax
import jax.numpy as jnp
from jax.experimental import pallas as pl
from jax.experimental.pallas import tpu as pltpu

NEIGHBOURS = 8


def _gcn_body(x_ref, ut_ref, vt_ref, ub_ref, vb_ref, o_ref, *, t, nj):
    xb = x_ref[0]
    xg = [xb[:, jj, :] for jj in range(nj)]

    sims = [
        jax.lax.dot_general(
            xj, xj, (((1,), (1,)), ((), ())),
            preferred_element_type=jnp.float32,
        )
        for xj in xg
    ]
    sim3 = jnp.stack(sims, axis=0)

    neg = jnp.float32(-jnp.inf)
    work = sim3
    thr = None
    for it in range(NEIGHBOURS):
        thr = jnp.max(work, axis=1, keepdims=True)
        if it < NEIGHBOURS - 1:
            work = jnp.where(work >= thr, neg, work)

    bf = jnp.bfloat16
    ones = jnp.ones((t, 128), bf)
    vt = vt_ref[...].astype(bf)
    ut = ut_ref[...].astype(bf)
    vb = vb_ref[...]
    ub = ub_ref[...]
    outs = []
    for i in range(nj):
        thr_col = jnp.transpose(thr[i], (1, 0))
        adj = (sims[i] >= thr_col).astype(bf)
        deg = jnp.dot(adj, ones, preferred_element_type=jnp.float32)[:, 0:1]
        dinv = jax.lax.rsqrt(deg)
        xgb = xg[i].astype(bf)
        vx = jnp.dot(xgb, vt, preferred_element_type=jnp.float32) + vb
        agg = (
            jnp.dot(adj, (vx * dinv).astype(bf),
                    preferred_element_type=jnp.float32)
            * dinv
        )
        ux = jnp.dot(xgb, ut, preferred_element_type=jnp.float32) + ub
        outs.append(jnp.maximum(xg[i] + agg + ux, 0.0))
    o_ref[0] = jnp.stack(outs, axis=1)


@jax.jit
def kernel(x, U_w, U_b, V_w, V_b):
    b, t, j, c = x.shape
    ut = U_w.T
    vt = V_w.T
    ub = U_b.reshape(1, c)
    vb = V_b.reshape(1, c)

    return pl.pallas_call(
        functools.partial(_gcn_body, t=t, nj=j),
        grid=(b,),
        in_specs=[
            pl.BlockSpec((1, t, j, c), lambda bi: (bi, 0, 0, 0)),
            pl.BlockSpec((c, c), lambda bi: (0, 0)),
            pl.BlockSpec((c, c), lambda bi: (0, 0)),
            pl.BlockSpec((1, c), lambda bi: (0, 0)),
            pl.BlockSpec((1, c), lambda bi: (0, 0)),
        ],
        out_specs=pl.BlockSpec((1, t, j, c), lambda bi: (bi, 0, 0, 0)),
        out_shape=jax.ShapeDtypeStruct((b, t, j, c), jnp.float32),
        compiler_params=pltpu.CompilerParams(
            dimension_semantics=("arbitrary",),
        ),
    )(x, ut, vt, ub, vb)


# speedup vs baseline: 1.4196x; 1.4196x over previous
import functools

import jax
import jax.numpy as jnp
from jax.experimental import pallas as pl
from jax.experimental.pallas import tpu as pltpu

NEIGHBOURS = 8


def _gcn_body(x_ref, ut_ref, vt_ref, ub_ref, vb_ref, o_ref, *, t, nj):
    xb = x_ref[0]
    xt = jnp.transpose(xb, (1, 0, 2))
    xg = [xt[jj] for jj in range(nj)]

    sims = [
        jax.lax.dot_general(
            xj, xj, (((1,), (1,)), ((), ())),
            preferred_element_type=jnp.float32,
        )
        for xj in xg
    ]
    sim3 = jnp.stack(sims, axis=0)

    neg = jnp.float32(-jnp.inf)
    work = sim3
    thr = None
    for it in range(NEIGHBOURS):
        thr = jnp.max(work, axis=1, keepdims=True)
        if it < NEIGHBOURS - 1:
            work = jnp.where(work >= thr, neg, work)

    ones = jnp.ones((t, 128), jnp.float32)
    vt = vt_ref[...]
    ut = ut_ref[...]
    vb = vb_ref[...]
    ub = ub_ref[...]
    outs = []
    for i in range(nj):
        thr_col = jnp.transpose(thr[i], (1, 0))
        adj = (sims[i] >= thr_col).astype(jnp.float32)
        deg = jnp.dot(adj, ones, preferred_element_type=jnp.float32)[:, 0:1]
        dinv = jax.lax.rsqrt(deg)
        vx = jnp.dot(xg[i], vt, preferred_element_type=jnp.float32) + vb
        agg = (
            jnp.dot(adj, vx * dinv, preferred_element_type=jnp.float32)
            * dinv
        )
        ux = jnp.dot(xg[i], ut, preferred_element_type=jnp.float32) + ub
        outs.append(jnp.maximum(xg[i] + agg + ux, 0.0))
    o_ref[0] = jnp.transpose(jnp.stack(outs, axis=0), (1, 0, 2))


@jax.jit
def kernel(x, U_w, U_b, V_w, V_b):
    b, t, j, c = x.shape
    ut = U_w.T
    vt = V_w.T
    ub = U_b.reshape(1, c)
    vb = V_b.reshape(1, c)

    return pl.pallas_call(
        functools.partial(_gcn_body, t=t, nj=j),
        grid=(b,),
        in_specs=[
            pl.BlockSpec((1, t, j, c), lambda bi: (bi, 0, 0, 0)),
            pl.BlockSpec((c, c), lambda bi: (0, 0)),
            pl.BlockSpec((c, c), lambda bi: (0, 0)),
            pl.BlockSpec((1, c), lambda bi: (0, 0)),
            pl.BlockSpec((1, c), lambda bi: (0, 0)),
        ],
        out_specs=pl.BlockSpec((1, t, j, c), lambda bi: (bi, 0, 0, 0)),
        out_shape=jax.ShapeDtypeStruct((b, t, j, c), jnp.float32),
        compiler_params=pltpu.CompilerParams(
            dimension_semantics=("arbitrary",),
        ),
    )(x, ut, vt, ub, vb)


# speedup vs baseline: 1.5226x; 1.0726x over previous
import functools

import jax
import jax.numpy as jnp
from jax.experimental import pallas as pl
from jax.experimental.pallas import tpu as pltpu

NEIGHBOURS = 8


def _gcn_body(x_ref, ut_ref, vt_ref, ub_ref, vb_ref, o_ref, *, t, nj):
    xb = x_ref[0]
    xt = jnp.transpose(xb, (1, 0, 2))
    xg = [xt[jj] for jj in range(nj)]

    sims = [
        jax.lax.dot_general(
            xj, xj, (((1,), (1,)), ((), ())),
            preferred_element_type=jnp.float32,
        )
        for xj in xg
    ]
    sim3 = jnp.stack(sims, axis=0)

    neg = jnp.float32(-jnp.inf)
    work = sim3
    thr = None
    for it in range(NEIGHBOURS):
        thr = jnp.max(work, axis=1, keepdims=True)
        if it < NEIGHBOURS - 1:
            work = jnp.where(work >= thr, neg, work)

    adjt3 = (sim3 >= thr).astype(jnp.float32)
    deg3 = jnp.sum(adjt3, axis=1, keepdims=True)
    dinv3 = jax.lax.rsqrt(jnp.transpose(deg3, (0, 2, 1)))

    vt = vt_ref[...]
    ut = ut_ref[...]
    vb = vb_ref[...]
    ub = ub_ref[...]
    outs = []
    for i in range(nj):
        dinv = dinv3[i]
        vx = jnp.dot(xg[i], vt, preferred_element_type=jnp.float32) + vb
        agg = jax.lax.dot_general(
            adjt3[i], vx * dinv, (((0,), (0,)), ((), ())),
            preferred_element_type=jnp.float32,
        ) * dinv
        ux = jnp.dot(xg[i], ut, preferred_element_type=jnp.float32) + ub
        outs.append(jnp.maximum(xg[i] + agg + ux, 0.0))
    o_ref[0] = jnp.transpose(jnp.stack(outs, axis=0), (1, 0, 2))


@jax.jit
def kernel(x, U_w, U_b, V_w, V_b):
    b, t, j, c = x.shape
    ut = U_w.T
    vt = V_w.T
    ub = U_b.reshape(1, c)
    vb = V_b.reshape(1, c)

    return pl.pallas_call(
        functools.partial(_gcn_body, t=t, nj=j),
        grid=(b,),
        in_specs=[
            pl.BlockSpec((1, t, j, c), lambda bi: (bi, 0, 0, 0)),
            pl.BlockSpec((c, c), lambda bi: (0, 0)),
            pl.BlockSpec((c, c), lambda bi: (0, 0)),
            pl.BlockSpec((1, c), lambda bi: (0, 0)),
            pl.BlockSpec((1, c), lambda bi: (0, 0)),
        ],
        out_specs=pl.BlockSpec((1, t, j, c), lambda bi: (bi, 0, 0, 0)),
        out_shape=jax.ShapeDtypeStruct((b, t, j, c), jnp.float32),
        compiler_params=pltpu.CompilerParams(
            dimension_semantics=("arbitrary",),
        ),
    )(x, ut, vt, ub, vb)


# speedup vs baseline: 1.6618x; 1.0914x over previous
import functools

import jax
import jax.numpy as jnp
from jax.experimental import pallas as pl
from jax.experimental.pallas import tpu as pltpu

NEIGHBOURS = 8


def _gcn_body(x_ref, ut_ref, vt_ref, ub_ref, vb_ref, o_ref, *, t, nj):
    xb = x_ref[0]
    xg = [xb[jj] for jj in range(nj)]

    sims = [
        jax.lax.dot_general(
            xj, xj, (((1,), (1,)), ((), ())),
            preferred_element_type=jnp.float32,
        )
        for xj in xg
    ]
    sim3 = jnp.stack(sims, axis=0)

    neg = jnp.float32(-jnp.inf)
    work = sim3
    thr = None
    for it in range(NEIGHBOURS):
        thr = jnp.max(work, axis=1, keepdims=True)
        if it < NEIGHBOURS - 1:
            work = jnp.where(work >= thr, neg, work)

    adjt3 = (sim3 >= thr).astype(jnp.float32)
    deg3 = jnp.sum(adjt3, axis=1, keepdims=True)
    dinv3 = jax.lax.rsqrt(jnp.transpose(deg3, (0, 2, 1)))

    vt = vt_ref[...]
    ut = ut_ref[...]
    vb = vb_ref[...]
    ub = ub_ref[...]
    outs = []
    for i in range(nj):
        dinv = dinv3[i]
        vx = jnp.dot(xg[i], vt, preferred_element_type=jnp.float32) + vb
        agg = jax.lax.dot_general(
            adjt3[i], vx * dinv, (((0,), (0,)), ((), ())),
            preferred_element_type=jnp.float32,
        ) * dinv
        ux = jnp.dot(xg[i], ut, preferred_element_type=jnp.float32) + ub
        outs.append(jnp.maximum(xg[i] + agg + ux, 0.0))
    o_ref[0] = jnp.stack(outs, axis=0)


@jax.jit
def kernel(x, U_w, U_b, V_w, V_b):
    b, t, j, c = x.shape
    xt = jnp.transpose(x, (0, 2, 1, 3))
    ut = U_w.T
    vt = V_w.T
    ub = U_b.reshape(1, c)
    vb = V_b.reshape(1, c)

    outt = pl.pallas_call(
        functools.partial(_gcn_body, t=t, nj=j),
        grid=(b,),
        in_specs=[
            pl.BlockSpec((1, j, t, c), lambda bi: (bi, 0, 0, 0)),
            pl.BlockSpec((c, c), lambda bi: (0, 0)),
            pl.BlockSpec((c, c), lambda bi: (0, 0)),
            pl.BlockSpec((1, c), lambda bi: (0, 0)),
            pl.BlockSpec((1, c), lambda bi: (0, 0)),
        ],
        out_specs=pl.BlockSpec((1, j, t, c), lambda bi: (bi, 0, 0, 0)),
        out_shape=jax.ShapeDtypeStruct((b, j, t, c), jnp.float32),
        compiler_params=pltpu.CompilerParams(
            dimension_semantics=("arbitrary",),
        ),
    )(xt, ut, vt, ub, vb)
    return jnp.transpose(outt, (0, 2, 1, 3))


# speedup vs baseline: 1.6672x; 1.0033x over previous
import functools

import jax
import jax.numpy as jnp
from jax.experimental import pallas as pl
from jax.experimental.pallas import tpu as pltpu

NEIGHBOURS = 8


def _gcn_body(x_ref, ut_ref, vt_ref, ub_ref, vb_ref, o_ref, *, t, nj):
    xb = x_ref[0]
    xg = [xb[jj] for jj in range(nj)]

    sims = [
        jax.lax.dot_general(
            xj, xj, (((1,), (1,)), ((), ())),
            preferred_element_type=jnp.float32,
        )
        for xj in xg
    ]
    sim3 = jnp.stack(sims, axis=0)

    neg = jnp.float32(-jnp.inf)
    work = sim3
    thr = None
    for it in range(NEIGHBOURS):
        thr = jnp.max(work, axis=1, keepdims=True)
        if it < NEIGHBOURS - 1:
            work = jnp.where(work >= thr, neg, work)

    adjt3 = (sim3 >= thr).astype(jnp.float32)
    deg3 = jnp.sum(adjt3, axis=1, keepdims=True)
    dinv3 = jax.lax.rsqrt(jnp.transpose(deg3, (0, 2, 1)))

    vt = vt_ref[...]
    ut = ut_ref[...]
    vb = vb_ref[...]
    ub = ub_ref[...]
    outs = []
    for i in range(nj):
        dinv = dinv3[i]
        vx = jnp.dot(xg[i], vt, preferred_element_type=jnp.float32) + vb
        agg = jax.lax.dot_general(
            adjt3[i], vx * dinv, (((0,), (0,)), ((), ())),
            preferred_element_type=jnp.float32,
        ) * dinv
        ux = jnp.dot(xg[i], ut, preferred_element_type=jnp.float32) + ub
        outs.append(jnp.maximum(xg[i] + agg + ux, 0.0))
    o_ref[0] = jnp.stack(outs, axis=0)


@jax.jit
def kernel(x, U_w, U_b, V_w, V_b):
    b, t, j, c = x.shape
    xt = jnp.transpose(x, (0, 2, 1, 3))
    ut = U_w.T
    vt = V_w.T
    ub = U_b.reshape(1, c)
    vb = V_b.reshape(1, c)

    outt = pl.pallas_call(
        functools.partial(_gcn_body, t=t, nj=j),
        grid=(b,),
        in_specs=[
            pl.BlockSpec((1, j, t, c), lambda bi: (bi, 0, 0, 0)),
            pl.BlockSpec((c, c), lambda bi: (0, 0)),
            pl.BlockSpec((c, c), lambda bi: (0, 0)),
            pl.BlockSpec((1, c), lambda bi: (0, 0)),
            pl.BlockSpec((1, c), lambda bi: (0, 0)),
        ],
        out_specs=pl.BlockSpec((1, j, t, c), lambda bi: (bi, 0, 0, 0)),
        out_shape=jax.ShapeDtypeStruct((b, j, t, c), jnp.float32),
        compiler_params=pltpu.CompilerParams(
            dimension_semantics=("parallel",),
        ),
    )(xt, ut, vt, ub, vb)
    return jnp.transpose(outt, (0, 2, 1, 3))
